# baseline (speedup 1.0000x reference)
import jax
import jax.numpy as jnp
from jax.experimental import pallas as pl
from jax.experimental.pallas import tpu as pltpu

_BM = 512
_KS = 2


def _gate_kernel(x_ref, w_ref, o_ref):
    part = jax.lax.dot_general(
        x_ref[...], w_ref[...], (((1,), (1,)), ((), ())),
        precision=jax.lax.Precision.DEFAULT,
        preferred_element_type=jnp.float32)

    @pl.when(pl.program_id(1) == 0)
    def _():
        o_ref[...] = part

    @pl.when(pl.program_id(1) != 0)
    def _():
        o_ref[...] += part


def kernel(hidden_states, weight):
    m, k = hidden_states.shape
    e = weight.shape[0]
    kc = k // _KS
    return pl.pallas_call(
        _gate_kernel,
        grid=(m // _BM, _KS),
        in_specs=[
            pl.BlockSpec((_BM, kc), lambda i, j: (i, j)),
            pl.BlockSpec((e, kc), lambda i, j: (0, j)),
        ],
        out_specs=pl.BlockSpec((_BM, e), lambda i, j: (i, 0)),
        out_shape=jax.ShapeDtypeStruct((m, e), jnp.float32),
        compiler_params=pltpu.CompilerParams(
            dimension_semantics=(pltpu.PARALLEL, pltpu.ARBITRARY),
            disable_bounds_checks=True,
            skip_device_barrier=True),
    )(hidden_states, weight)


# speedup vs baseline: 1.2758x; 1.2758x over previous
import jax
import jax.numpy as jnp
from jax.experimental import pallas as pl
from jax.experimental.pallas import tpu as pltpu

_BM = 512


def _gate_kernel(x_ref, w_ref, o_ref):
    o_ref[...] = jax.lax.dot_general(
        x_ref[...], w_ref[...], (((1,), (1,)), ((), ())),
        precision=jax.lax.Precision.DEFAULT,
        preferred_element_type=jnp.float32)


def kernel(hidden_states, weight):
    m, k = hidden_states.shape
    e = weight.shape[0]
    return pl.pallas_call(
        _gate_kernel,
        grid=(m // _BM,),
        in_specs=[
            pl.BlockSpec((_BM, k), lambda i: (i, 0)),
            pl.BlockSpec((e, k), lambda i: (0, 0)),
        ],
        out_specs=pl.BlockSpec((_BM, e), lambda i: (i, 0)),
        out_shape=jax.ShapeDtypeStruct((m, e), jnp.float32),
        compiler_params=pltpu.CompilerParams(
            dimension_semantics=(pltpu.PARALLEL,),
            disable_bounds_checks=True,
            skip_device_barrier=True),
    )(hidden_states, weight)
